# Initial kernel scaffold; baseline (speedup 1.0000x reference)
import jax
import jax.numpy as jnp
from jax.experimental import pallas as pl


def kernel(keyword_lists, keyword_lengths, embedding_weight):
    raise NotImplementedError("write your pallas kernel here")


# speedup vs baseline: 11.0845x; 11.0845x over previous
import functools

import jax
import jax.numpy as jnp
from jax import lax
from jax.experimental import pallas as pl
from jax.experimental.pallas import tpu as pltpu
from jax.experimental.pallas import tpu_sc as plsc

_VOCAB1 = 100001
_D = 128
_B = 4096
_L = 200
_LC = 100
_NC = 2
_NS = 16
_NW = _NC * _NS
_BPW = _B // _NW
_NG = _BPW // 16
_NV = _D // 16


def _sc_pooled_lookup(kw2, lens, table):
  mesh = plsc.VectorSubcoreMesh(
      core_axis_name="c", subcore_axis_name="s", num_cores=_NC,
      num_subcores=_NS)

  @functools.partial(
      pl.kernel,
      out_type=jax.ShapeDtypeStruct((_B, _D), jnp.float32),
      mesh=mesh,
      scratch_types=[
          pltpu.VMEM((2 * _BPW, _LC), jnp.int32),
          pltpu.VMEM((2, _L, _D), jnp.float32),
          pltpu.VMEM((_BPW, _D), jnp.float32),
          pltpu.VMEM((_BPW,), jnp.int32),
          pltpu.VMEM((_BPW,), jnp.float32),
          pltpu.SemaphoreType.DMA,
          pltpu.SemaphoreType.DMA,
      ],
  )
  def kernel_body(kw_hbm, len_hbm, table_hbm, out_hbm,
                  idx_v, rows_v, out_v, len_v, recip_v, sem0, sem1):
    sems = (sem0, sem1)
    wid = lax.axis_index("s") * _NC + lax.axis_index("c")
    base = wid * _BPW

    pltpu.sync_copy(kw_hbm.at[pl.ds(base * 2, 2 * _BPW)], idx_v)
    pltpu.sync_copy(len_hbm.at[pl.ds(base, _BPW)], len_v)

    for j in range(_NG):
      lv = len_v[pl.ds(j * 16, 16)]
      lf = jnp.maximum(lv.astype(jnp.float32), 1.0)
      recip_v[pl.ds(j * 16, 16)] = 1.0 / lf

    def issue_gather(b, buf):
      pltpu.async_copy(table_hbm.at[idx_v.at[2 * b]],
                       rows_v.at[buf, pl.ds(0, _LC)], sems[buf])
      pltpu.async_copy(table_hbm.at[idx_v.at[2 * b + 1]],
                       rows_v.at[buf, pl.ds(_LC, _LC)], sems[buf])

    def wait_gather(buf):
      pltpu.make_async_copy(table_hbm.at[pl.ds(0, _L)], rows_v.at[buf],
                            sems[buf]).wait()

    issue_gather(jnp.int32(0), 0)

    def group_body(j, carry):
      rchunk = recip_v[pl.ds(j * 16, 16)]
      for k in range(16):
        b = j * 16 + k
        buf = k % 2

        wait_gather(buf)
        nb = b + 1

        @pl.when(nb < _BPW)
        def _():
          issue_gather(nb, (k + 1) % 2)

        def red_body(r, acc):
          return tuple(acc[d] + rows_v[buf, r, pl.ds(d * 16, 16)]
                       for d in range(_NV))

        acc = lax.fori_loop(
            0, _L, red_body,
            tuple(jnp.zeros((16,), jnp.float32) for _ in range(_NV)))

        rk = jnp.broadcast_to(lax.slice(rchunk, (k,), (k + 1,)), (16,))
        for d in range(_NV):
          out_v[b, pl.ds(d * 16, 16)] = acc[d] * rk
      return carry

    lax.fori_loop(0, _NG, group_body, 0)

    pltpu.sync_copy(out_v, out_hbm.at[pl.ds(base, _BPW)])

  return kernel_body(kw2, lens, table)


@jax.jit
def kernel(keyword_lists, keyword_lengths, embedding_weight):
  kw2 = keyword_lists.reshape(_B * 2, _LC)
  lens = keyword_lengths.reshape(_B)
  return _sc_pooled_lookup(kw2, lens, embedding_weight)


# speedup vs baseline: 13.7491x; 1.2404x over previous
import functools

import jax
import jax.numpy as jnp
from jax import lax
from jax.experimental import pallas as pl
from jax.experimental.pallas import tpu as pltpu
from jax.experimental.pallas import tpu_sc as plsc

_VOCAB1 = 100001
_D = 128
_B = 4096
_L = 200
_LC = 100
_NC = 2
_NS = 16
_NW = _NC * _NS
_BPW = _B // _NW
_NG = _BPW // 16
_NV = _D // 16


def _sc_pooled_lookup(kw2, lens, table):
  mesh = plsc.VectorSubcoreMesh(
      core_axis_name="c", subcore_axis_name="s", num_cores=_NC,
      num_subcores=_NS)

  @functools.partial(
      pl.kernel,
      out_type=jax.ShapeDtypeStruct((_B, _D), jnp.float32),
      mesh=mesh,
      scratch_types=[
          pltpu.VMEM((2 * _BPW, _LC), jnp.int32),
          pltpu.VMEM((2, _L, _D), jnp.float32),
          pltpu.VMEM((_BPW, _D), jnp.float32),
          pltpu.VMEM((_BPW,), jnp.int32),
          pltpu.VMEM((_BPW,), jnp.float32),
          pltpu.SemaphoreType.DMA,
          pltpu.SemaphoreType.DMA,
      ],
  )
  def kernel_body(kw_hbm, len_hbm, table_hbm, out_hbm,
                  idx_v, rows_v, out_v, len_v, recip_v, sem0, sem1):
    sems = (sem0, sem1)
    wid = lax.axis_index("s") * _NC + lax.axis_index("c")
    base = wid * _BPW

    pltpu.sync_copy(kw_hbm.at[pl.ds(base * 2, 2 * _BPW)], idx_v)
    pltpu.sync_copy(len_hbm.at[pl.ds(base, _BPW)], len_v)

    for j in range(_NG):
      lv = len_v[pl.ds(j * 16, 16)]
      lf = jnp.maximum(lv.astype(jnp.float32), 1.0)
      recip_v[pl.ds(j * 16, 16)] = 1.0 / lf

    def issue_gather(b, buf):
      pltpu.async_copy(table_hbm.at[idx_v.at[2 * b]],
                       rows_v.at[buf, pl.ds(0, _LC)], sems[buf])
      pltpu.async_copy(table_hbm.at[idx_v.at[2 * b + 1]],
                       rows_v.at[buf, pl.ds(_LC, _LC)], sems[buf])

    def wait_gather(buf):
      pltpu.make_async_copy(table_hbm.at[pl.ds(0, _L)], rows_v.at[buf],
                            sems[buf]).wait()

    issue_gather(jnp.int32(0), 0)

    def group_body(j, carry):
      rchunk = recip_v[pl.ds(j * 16, 16)]
      for k in range(16):
        b = j * 16 + k
        buf = k % 2

        nb = b + 1

        @pl.when(nb < _BPW)
        def _():
          issue_gather(nb, (k + 1) % 2)

        wait_gather(buf)

        def red_body(r, acc):
          return tuple(acc[d] + rows_v[buf, r, pl.ds(d * 16, 16)]
                       for d in range(_NV))

        acc = lax.fori_loop(
            0, _L, red_body,
            tuple(jnp.zeros((16,), jnp.float32) for _ in range(_NV)))

        rk = jnp.broadcast_to(lax.slice(rchunk, (k,), (k + 1,)), (16,))
        for d in range(_NV):
          out_v[b, pl.ds(d * 16, 16)] = acc[d] * rk
      return carry

    lax.fori_loop(0, _NG, group_body, 0)

    pltpu.sync_copy(out_v, out_hbm.at[pl.ds(base, _BPW)])

  return kernel_body(kw2, lens, table)


@jax.jit
def kernel(keyword_lists, keyword_lengths, embedding_weight):
  kw2 = keyword_lists.reshape(_B * 2, _LC)
  lens = keyword_lengths.reshape(_B)
  return _sc_pooled_lookup(kw2, lens, embedding_weight)
